# baseline (speedup 1.0000x reference)
import functools

import jax
import jax.numpy as jnp
from jax import lax
from jax.experimental import pallas as pl
from jax.experimental.pallas import tpu as pltpu
from jax.experimental.pallas import tpu_sc as plsc

_NC = 2
_NS = 16
_NW = _NC * _NS
_MROWS = 40320
_ROWS_TOTAL = 4 * _MROWS
_RPW = _ROWS_TOTAL // _NW
_CH = 360
_NCH = _RPW // _CH
_FEAT = 100
_LANES = 16
_COLS = (0, 16, 32, 48, 64, 80, 84)


def _sc_body(x_hbm, m_hbm, o_hbm,
             buf0, buf1, mbuf0, mbuf1,
             is0, is1, ims0, ims1, os0, os1):
    wid = lax.axis_index("s") * _NC + lax.axis_index("c")
    base = wid * _RPW
    mbase = lax.rem(base, _MROWS) * _LANES
    bufs = (buf0, buf1)
    mbufs = (mbuf0, mbuf1)
    isems = (is0, is1)
    imsems = (ims0, ims1)
    osems = (os0, os1)

    def in_cp(t, b):
        return pltpu.make_async_copy(
            x_hbm.at[pl.ds(base + t * _CH, _CH), :], bufs[b], isems[b]
        )

    def min_cp(t, b):
        return pltpu.make_async_copy(
            m_hbm.at[pl.ds(mbase + t * _CH * _LANES, _CH * _LANES)],
            mbufs[b],
            imsems[b],
        )

    def out_cp(t, b):
        return pltpu.make_async_copy(
            bufs[b], o_hbm.at[pl.ds(base + t * _CH, _CH), :], osems[b]
        )

    in_cp(0, 0).start()
    min_cp(0, 0).start()

    for t in range(_NCH):
        b = t % 2
        buf = bufs[b]
        mbuf = mbufs[b]
        in_cp(t, b).wait()
        min_cp(t, b).wait()

        def row_fn(r, carry, buf=buf, mbuf=mbuf):
            mval = mbuf[pl.ds(r * _LANES, _LANES)]
            vals = [buf[r, pl.ds(c, _LANES)] for c in _COLS]
            for c, v in zip(_COLS, vals):
                buf[r, pl.ds(c, _LANES)] = v * mval
            return carry

        lax.fori_loop(0, _CH, row_fn, 0)

        out_cp(t, b).start()
        if t + 1 < _NCH:
            if t >= 1:
                out_cp(t - 1, 1 - b).wait()
            in_cp(t + 1, 1 - b).start()
            min_cp(t + 1, 1 - b).start()
    out_cp(_NCH - 1, (_NCH - 1) % 2).wait()


def kernel(x, mask, dim):
    del dim
    x2 = x.reshape(_ROWS_TOTAL, _FEAT)
    mask16 = jnp.repeat(mask.astype(jnp.float32), _LANES)
    mesh = plsc.VectorSubcoreMesh(core_axis_name="c", subcore_axis_name="s")
    k = functools.partial(
        pl.kernel,
        mesh=mesh,
        out_type=jax.ShapeDtypeStruct((_ROWS_TOTAL, _FEAT), jnp.float32),
        scratch_types=[
            pltpu.VMEM((_CH, _FEAT), jnp.float32),
            pltpu.VMEM((_CH, _FEAT), jnp.float32),
            pltpu.VMEM((_CH * _LANES,), jnp.float32),
            pltpu.VMEM((_CH * _LANES,), jnp.float32),
            pltpu.SemaphoreType.DMA,
            pltpu.SemaphoreType.DMA,
            pltpu.SemaphoreType.DMA,
            pltpu.SemaphoreType.DMA,
            pltpu.SemaphoreType.DMA,
            pltpu.SemaphoreType.DMA,
        ],
    )(_sc_body)
    out = k(x2, mask16)
    return out.reshape(x.shape)


# speedup vs baseline: 1.1158x; 1.1158x over previous
import functools

import jax
import jax.numpy as jnp
from jax import lax
from jax.experimental import pallas as pl
from jax.experimental.pallas import tpu as pltpu
from jax.experimental.pallas import tpu_sc as plsc

_NC = 2
_NS = 16
_NW = _NC * _NS
_MROWS = 40320
_ROWS_TOTAL = 4 * _MROWS
_RPW = _ROWS_TOTAL // _NW
_CH = 240
_NCH = _RPW // _CH
_NB = 3
_FEAT = 100
_LANES = 16
_COLS = (0, 16, 32, 48, 64, 80, 84)


def _sc_body(x_hbm, m_hbm, o_hbm,
             buf0, buf1, buf2, mbuf0, mbuf1, mbuf2,
             is0, is1, is2, ims0, ims1, ims2, os0, os1, os2):
    wid = lax.axis_index("s") * _NC + lax.axis_index("c")
    base = wid * _RPW
    mbase = lax.rem(base, _MROWS) * _LANES
    bufs = (buf0, buf1, buf2)
    mbufs = (mbuf0, mbuf1, mbuf2)
    isems = (is0, is1, is2)
    imsems = (ims0, ims1, ims2)
    osems = (os0, os1, os2)

    def in_cp(t, b):
        return pltpu.make_async_copy(
            x_hbm.at[pl.ds(base + t * _CH, _CH), :], bufs[b], isems[b]
        )

    def min_cp(t, b):
        return pltpu.make_async_copy(
            m_hbm.at[pl.ds(mbase + t * _CH * _LANES, _CH * _LANES)],
            mbufs[b],
            imsems[b],
        )

    def out_cp(t, b):
        return pltpu.make_async_copy(
            bufs[b], o_hbm.at[pl.ds(base + t * _CH, _CH), :], osems[b]
        )

    for t in range(2):
        in_cp(t, t).start()
        min_cp(t, t).start()

    for t in range(_NCH):
        b = t % _NB
        buf = bufs[b]
        mbuf = mbufs[b]
        in_cp(t, b).wait()
        min_cp(t, b).wait()

        def row_fn(r, carry, buf=buf, mbuf=mbuf):
            mval = mbuf[pl.ds(r * _LANES, _LANES)]
            vals = [buf[r, pl.ds(c, _LANES)] for c in _COLS]
            for c, v in zip(_COLS, vals):
                buf[r, pl.ds(c, _LANES)] = v * mval
            return carry

        lax.fori_loop(0, _CH, row_fn, 0)

        out_cp(t, b).start()
        if t + 2 < _NCH:
            nb = (t + 2) % _NB
            if t >= 1:
                out_cp(t - 1, nb).wait()
            in_cp(t + 2, nb).start()
            min_cp(t + 2, nb).start()
    for t in range(_NCH - 3, _NCH):
        out_cp(t, t % _NB).wait()


def kernel(x, mask, dim):
    del dim
    x2 = x.reshape(_ROWS_TOTAL, _FEAT)
    mask16 = jnp.repeat(mask.astype(jnp.float32), _LANES)
    mesh = plsc.VectorSubcoreMesh(core_axis_name="c", subcore_axis_name="s")
    k = functools.partial(
        pl.kernel,
        mesh=mesh,
        out_type=jax.ShapeDtypeStruct((_ROWS_TOTAL, _FEAT), jnp.float32),
        scratch_types=[
            pltpu.VMEM((_CH, _FEAT), jnp.float32),
            pltpu.VMEM((_CH, _FEAT), jnp.float32),
            pltpu.VMEM((_CH, _FEAT), jnp.float32),
            pltpu.VMEM((_CH * _LANES,), jnp.float32),
            pltpu.VMEM((_CH * _LANES,), jnp.float32),
            pltpu.VMEM((_CH * _LANES,), jnp.float32),
            pltpu.SemaphoreType.DMA,
            pltpu.SemaphoreType.DMA,
            pltpu.SemaphoreType.DMA,
            pltpu.SemaphoreType.DMA,
            pltpu.SemaphoreType.DMA,
            pltpu.SemaphoreType.DMA,
            pltpu.SemaphoreType.DMA,
            pltpu.SemaphoreType.DMA,
            pltpu.SemaphoreType.DMA,
        ],
    )(_sc_body)
    out = k(x2, mask16)
    return out.reshape(x.shape)
